# baseline (speedup 1.0000x reference)
import jax
import jax.numpy as jnp
from jax.experimental import pallas as pl
from jax.experimental.pallas import tpu as pltpu
from jax.experimental.pallas import tpu_sc as plsc

_WINDOW = 128
_BB = 32
_CB = 128


def _sc_repack(table_t, v, d):
    vpad = ((v + _CB - 1) // _CB) * _CB
    mesh = plsc.VectorSubcoreMesh(core_axis_name="core", subcore_axis_name="subcore")

    @pl.kernel(
        out_type=jax.ShapeDtypeStruct((vpad // 2, 2 * d), jnp.float32),
        mesh=mesh,
        compiler_params=pltpu.CompilerParams(
            needs_layout_passes=False, disable_bounds_checks=True
        ),
    )
    def repack_kernel(t_hbm, o_hbm):
        def body(t_vmem, o_vmem):
            lane = jax.lax.broadcasted_iota(jnp.int32, (16,), 0)

            @pl.loop(0, _CB // 2)
            def _(q):
                for c in range(2 * d // 16):
                    col = 2 * q + (16 * c) // d
                    feat = lane + (16 * c) % d
                    vals = plsc.load_gather(t_vmem, [feat, jnp.zeros((16,), jnp.int32) + col])
                    o_vmem[q, pl.ds(16 * c, 16)] = vals

        pltpu.emit_pipeline(
            body,
            grid=(vpad // _CB,),
            in_specs=[pl.BlockSpec((d, _CB), index_map=lambda i: (0, i))],
            out_specs=[pl.BlockSpec((_CB // 2, 2 * d), index_map=lambda i: (i, 0))],
            core_axis_name=("core", "subcore"),
            dimension_semantics=(pltpu.PARALLEL,),
        )(t_hbm, o_hbm)

    return repack_kernel(table_t), vpad


def _sc_gather(table_lin, idx3, n, d):
    nblk, _, rb = idx3.shape
    wpb = 2 * rb // _WINDOW
    half = _WINDOW // 2
    mesh = plsc.VectorSubcoreMesh(core_axis_name="core", subcore_axis_name="subcore")

    @pl.kernel(
        out_type=jax.ShapeDtypeStruct((n, d), jnp.float32),
        mesh=mesh,
        compiler_params=pltpu.CompilerParams(
            use_tc_tiling_on_sc=False, needs_layout_passes=False
        ),
        scratch_types=[pltpu.VMEM((_WINDOW,), jnp.int32)],
    )
    def gather_kernel(tab_hbm, i_hbm, o_hbm, ileave_ref):
        def body(i_vmem, o_vmem):
            lane = jax.lax.broadcasted_iota(jnp.int32, (16,), 0)
            zero = jnp.zeros((16,), jnp.int32)
            hsel = jax.lax.rem(lane, 2)
            tsel = jax.lax.shift_right_logical(lane, 1)
            for g in range(_WINDOW // 16):
                vals = plsc.load_gather(i_vmem, [zero, hsel, tsel + (8 * g)])
                ileave_ref[pl.ds(16 * g, 16)] = vals
            pltpu.sync_copy(tab_hbm.at[ileave_ref], o_vmem)

        pltpu.emit_pipeline(
            body,
            grid=(n // _WINDOW,),
            in_specs=[
                pl.BlockSpec(
                    (1, 2, half),
                    index_map=lambda w: (w // wpb, 0, w % wpb),
                )
            ],
            out_specs=[pl.BlockSpec((_WINDOW, d), index_map=lambda w: (w, 0))],
            core_axis_name=("core", "subcore"),
            dimension_semantics=(pltpu.PARALLEL,),
        )(i_hbm, o_hbm)

    return gather_kernel(table_lin, idx3)


def _tc_layernorm(emb2, gamma2, beta2, b, l, d):
    rb = _BB * l // 2
    hb = _BB // 2

    def ln_body(e_ref, g_ref, b_ref, o_ref):
        e = e_ref[...]
        lane = jax.lax.broadcasted_iota(jnp.int32, e.shape, 1)
        left = lane < d
        s_all = jnp.sum(e, axis=1, keepdims=True)
        s_l = jnp.sum(jnp.where(left, e, 0.0), axis=1, keepdims=True)
        sq = e * e
        q_all = jnp.sum(sq, axis=1, keepdims=True)
        q_l = jnp.sum(jnp.where(left, sq, 0.0), axis=1, keepdims=True)
        inv = 1.0 / d
        mean = jnp.where(left, s_l, s_all - s_l) * inv
        msq = jnp.where(left, q_l, q_all - q_l) * inv
        var = msq - mean * mean
        normed = (e - mean) * jax.lax.rsqrt(var + 1e-5) * g_ref[...] + b_ref[...]
        o_ref[0:hb, :, :] = normed[:, :d].reshape(hb, l, d)
        o_ref[hb : 2 * hb, :, :] = normed[:, d:].reshape(hb, l, d)

    return pl.pallas_call(
        ln_body,
        grid=(b // _BB,),
        in_specs=[
            pl.BlockSpec((rb, 2 * d), lambda i: (i, 0)),
            pl.BlockSpec((1, 2 * d), lambda i: (0, 0)),
            pl.BlockSpec((1, 2 * d), lambda i: (0, 0)),
        ],
        out_specs=pl.BlockSpec((_BB, l, d), lambda i: (i, 0, 0)),
        out_shape=jax.ShapeDtypeStruct((b, l, d), jnp.float32),
    )(emb2, gamma2, beta2)


def kernel(x, table, gamma, beta):
    b, l = x.shape
    v, d = table.shape
    n = b * l
    rb = _BB * l // 2
    repacked, vpad = _sc_repack(table.T, v, d)
    table_lin = repacked.reshape(vpad, d)
    idx3 = x.reshape(n).astype(jnp.int32).reshape(b // _BB, 2, rb)
    emb2 = _sc_gather(table_lin, idx3, n, d).reshape(n // 2, 2 * d)
    gamma2 = jnp.tile(gamma, 2).reshape(1, 2 * d)
    beta2 = jnp.tile(beta, 2).reshape(1, 2 * d)
    return _tc_layernorm(emb2, gamma2, beta2, b, l, d)


# speedup vs baseline: 1.8975x; 1.8975x over previous
import jax
import jax.numpy as jnp
from jax.experimental import pallas as pl
from jax.experimental.pallas import tpu as pltpu
from jax.experimental.pallas import tpu_sc as plsc

_WINDOW = 128


def _sc_gather(table, idx3, n, d):
    nblk, _, rb = idx3.shape
    wpb = 2 * rb // _WINDOW
    half = _WINDOW // 2
    mesh = plsc.VectorSubcoreMesh(core_axis_name="core", subcore_axis_name="subcore")

    @pl.kernel(
        out_type=jax.ShapeDtypeStruct((n, d), jnp.float32),
        mesh=mesh,
        compiler_params=pltpu.CompilerParams(
            use_tc_tiling_on_sc=False, needs_layout_passes=False
        ),
        scratch_types=[pltpu.VMEM((_WINDOW,), jnp.int32)],
    )
    def gather_kernel(tab_hbm, i_hbm, o_hbm, ileave_ref):
        def body(i_vmem, o_vmem):
            lane = jax.lax.broadcasted_iota(jnp.int32, (16,), 0)
            zero = jnp.zeros((16,), jnp.int32)
            hsel = jax.lax.rem(lane, 2)
            tsel = jax.lax.shift_right_logical(lane, 1)
            for g in range(_WINDOW // 16):
                vals = plsc.load_gather(i_vmem, [zero, hsel, tsel + (8 * g)])
                ileave_ref[pl.ds(16 * g, 16)] = vals
            pltpu.sync_copy(tab_hbm.at[ileave_ref], o_vmem)

        pltpu.emit_pipeline(
            body,
            grid=(n // _WINDOW,),
            in_specs=[
                pl.BlockSpec(
                    (1, 2, half),
                    index_map=lambda w: (w // wpb, 0, w % wpb),
                )
            ],
            out_specs=[pl.BlockSpec((_WINDOW, d), index_map=lambda w: (w, 0))],
            core_axis_name=("core", "subcore"),
            dimension_semantics=(pltpu.PARALLEL,),
        )(i_hbm, o_hbm)

    return gather_kernel(table, idx3)


def _tc_layernorm_t(embt, gamma, beta, b, l, d):
    hb = b // 2

    def ln_body(e_ref, g_ref, b_ref, o_ref):
        e3 = e_ref[...].reshape(2, d, hb)
        inv = 1.0 / d
        mean = jnp.sum(e3, axis=1, keepdims=True) * inv
        msq = jnp.sum(e3 * e3, axis=1, keepdims=True) * inv
        var = msq - mean * mean
        g3 = g_ref[...].reshape(1, d, 1)
        b3 = b_ref[...].reshape(1, d, 1)
        normed = (e3 - mean) * jax.lax.rsqrt(var + 1e-5) * g3 + b3
        o_ref[0, :, 0:hb] = normed[0]
        o_ref[0, :, hb : 2 * hb] = normed[1]

    return pl.pallas_call(
        ln_body,
        grid=(l,),
        in_specs=[
            pl.BlockSpec((2 * d, hb), lambda i: (0, i)),
            pl.BlockSpec((d, 1), lambda i: (0, 0)),
            pl.BlockSpec((d, 1), lambda i: (0, 0)),
        ],
        out_specs=pl.BlockSpec((1, d, b), lambda i: (i, 0, 0)),
        out_shape=jax.ShapeDtypeStruct((l, d, b), jnp.float32),
    )(embt, gamma.reshape(d, 1), beta.reshape(d, 1))


def kernel(x, table, gamma, beta):
    b, l = x.shape
    v, d = table.shape
    n = b * l
    idx3 = x.T.astype(jnp.int32).reshape(l, 2, b // 2)
    emb = _sc_gather(table, idx3, n, d)
    emb2 = emb.reshape(n // 2, 2 * d)
    embt = emb2.T
    out_t = _tc_layernorm_t(embt, gamma, beta, b, l, d)
    return out_t.transpose(2, 0, 1)
